# baseline (speedup 1.0000x reference)
import functools

import jax
import jax.numpy as jnp
from jax import lax
from jax.experimental import pallas as pl
from jax.experimental.pallas import tpu as pltpu
from jax.experimental.pallas import tpu_sc as plsc

B = 16384
F = 32
NC = 2
NS = 16
NW = NC * NS
BPW = B // NW


def _mf_body(uid_hbm, iid_hbm, p_hbm, q_hbm, ub_hbm, ib_hbm, out_hbm,
             uid_v, iid_v, pu_v, qi_v, bu_v, bi_v, out_v,
             sem_p, sem_q, sem_b):
  wid = lax.axis_index("s") * NC + lax.axis_index("c")
  base = wid * BPW

  pltpu.sync_copy(uid_hbm.at[pl.ds(base, BPW)], uid_v)
  pltpu.sync_copy(iid_hbm.at[pl.ds(base, BPW)], iid_v)

  cp_p = pltpu.async_copy(p_hbm.at[uid_v], pu_v, sem_p)
  cp_q = pltpu.async_copy(q_hbm.at[iid_v], qi_v, sem_q)
  cp_bu = pltpu.async_copy(ub_hbm.at[uid_v], bu_v, sem_b)
  cp_bi = pltpu.async_copy(ib_hbm.at[iid_v], bi_v, sem_b)
  cp_p.wait()
  cp_q.wait()

  lane = lax.iota(jnp.int32, 16)
  mask15 = lane == 15

  def row(b, _):
    pa = pu_v[b, pl.ds(0, 16)]
    pb = pu_v[b, pl.ds(16, 16)]
    qa = qi_v[b, pl.ds(0, 16)]
    qb = qi_v[b, pl.ds(16, 16)]
    prod = pa * qa + pb * qb
    c = plsc.cumsum(prod)
    idx = jnp.full((16,), b, jnp.int32)
    plsc.store_scatter(out_v, [idx], c, mask=mask15)
    return _

  lax.fori_loop(0, BPW, row, 0, unroll=8)

  cp_bu.wait()
  cp_bi.wait()

  def chunk(c, _):
    o = out_v[pl.ds(c * 16, 16)] + bu_v[pl.ds(c * 16, 16)] + bi_v[pl.ds(c * 16, 16)]
    out_v[pl.ds(c * 16, 16)] = o
    return _

  lax.fori_loop(0, BPW // 16, chunk, 0, unroll=4)

  pltpu.sync_copy(out_v, out_hbm.at[pl.ds(base, BPW)])


@jax.jit
def _mf(user_id, item_id, P, Q, ub_flat, ib_flat):
  mesh = plsc.VectorSubcoreMesh(core_axis_name="c", subcore_axis_name="s",
                                num_cores=NC, num_subcores=NS)
  return pl.kernel(
      _mf_body,
      out_type=jax.ShapeDtypeStruct((B,), jnp.float32),
      mesh=mesh,
      scratch_types=[
          pltpu.VMEM((BPW,), jnp.int32),
          pltpu.VMEM((BPW,), jnp.int32),
          pltpu.VMEM((BPW, F), jnp.float32),
          pltpu.VMEM((BPW, F), jnp.float32),
          pltpu.VMEM((BPW,), jnp.float32),
          pltpu.VMEM((BPW,), jnp.float32),
          pltpu.VMEM((BPW,), jnp.float32),
          pltpu.SemaphoreType.DMA,
          pltpu.SemaphoreType.DMA,
          pltpu.SemaphoreType.DMA,
      ],
      compiler_params=pltpu.CompilerParams(needs_layout_passes=False,
                                           use_tc_tiling_on_sc=False),
  )(user_id, item_id, P, Q, ub_flat, ib_flat)


def kernel(user_id, item_id, P, Q, user_bias, item_bias):
  return _mf(user_id, item_id, P, Q,
             user_bias.reshape(-1), item_bias.reshape(-1))
